# baseline (speedup 1.0000x reference)
import functools

import jax
import jax.numpy as jnp
from jax import lax
from jax.experimental import pallas as pl
from jax.experimental.pallas import tpu as pltpu
from jax.experimental.pallas import tpu_sc as plsc

E = 8
K = 2
T = 2048
D = 1024
DFF = 2816

BLK = 256
NB = (T * K) // BLK + E
P = NB * BLK
NSPLIT = 2
DFF_C = DFF // NSPLIT

NC = 2
NS = 16
NW = NC * NS
L = 16

_INTERPRET = False


def _gate_body(x_ref, gw_ref, e1_ref, e2_ref, w1_ref, w2_ref):
    x = x_ref[...]
    gl = jax.lax.dot_general(x, gw_ref[...], (((1,), (1,)), ((), ())))
    iota = jax.lax.broadcasted_iota(jnp.int32, gl.shape, 1)
    m1 = jnp.max(gl, axis=1, keepdims=True)
    a1 = jnp.min(jnp.where(gl == m1, iota, E), axis=1, keepdims=True)
    masked = jnp.where(iota == a1, -jnp.inf, gl)
    m2 = jnp.max(masked, axis=1, keepdims=True)
    a2 = jnp.min(jnp.where(masked == m2, iota, E), axis=1, keepdims=True)
    p1 = 1.0 / (1.0 + jnp.exp(m2 - m1))
    e1_ref[...] = a1[:, 0]
    e2_ref[...] = a2[:, 0]
    w1_ref[...] = p1[:, 0]
    w2_ref[...] = 1.0 - p1[:, 0]


def _gate(inputs, gate_w):
    nblk = T // BLK
    return pl.pallas_call(
        _gate_body,
        grid=(nblk,),
        in_specs=[
            pl.BlockSpec((BLK, D), lambda i: (i, 0)),
            pl.BlockSpec((E, D), lambda i: (0, 0)),
        ],
        out_specs=[
            pl.BlockSpec((BLK,), lambda i: (i,)),
            pl.BlockSpec((BLK,), lambda i: (i,)),
            pl.BlockSpec((BLK,), lambda i: (i,)),
            pl.BlockSpec((BLK,), lambda i: (i,)),
        ],
        out_shape=[
            jax.ShapeDtypeStruct((T,), jnp.int32),
            jax.ShapeDtypeStruct((T,), jnp.int32),
            jax.ShapeDtypeStruct((T,), jnp.float32),
            jax.ShapeDtypeStruct((T,), jnp.float32),
        ],
        interpret=_INTERPRET,
    )(inputs, gate_w)


RPW = (T * K) // NW
SLICE = P // NW


def _route_body(e1_hbm, e2_hbm, wa_hbm, wb_hbm,
                tok_hbm, w_hbm, pos_hbm, sinfo_hbm,
                eva_ref, evb_ref, evs_ref, wvs_ref,
                cnt_sh, tok_sh, w_sh,
                allcnt_ref, cntbuf_ref, prefix_ref,
                tokbuf, posbuf, outbuf_i, outbuf_f, sinfo_v):
    c = lax.axis_index("c")
    s = lax.axis_index("s")
    lane = lax.broadcasted_iota(jnp.int32, (L,), 0)
    zi = lane * 0
    full_last = zi + (L - 1)

    def oh(e):
        return 1 - jnp.minimum(jnp.abs(lane - e), 1)

    pltpu.sync_copy(e1_hbm.at[pl.ds(s * RPW, RPW)], eva_ref)
    pltpu.sync_copy(e2_hbm.at[pl.ds(s * RPW, RPW)], evb_ref)

    def count_range(ev_ref):
        cnt = zi
        for ch in range(RPW // L):
            ev = ev_ref[pl.ds(ch * L, L)]
            for e in range(E):
                mi = 1 - jnp.minimum(jnp.abs(ev - e), 1)
                pc = jnp.cumsum(mi)
                cnt = cnt + oh(e) * jnp.take(pc, full_last)
        return cnt

    cntbuf_ref[0, pl.ds(0, L)] = count_range(eva_ref)
    pltpu.sync_copy(cntbuf_ref, cnt_sh.at[pl.ds(s, 1)])
    cntbuf_ref[0, pl.ds(0, L)] = count_range(evb_ref)
    pltpu.sync_copy(cntbuf_ref, cnt_sh.at[pl.ds(s + NS, 1)])
    plsc.subcore_barrier()
    pltpu.sync_copy(cnt_sh, allcnt_ref)

    total = zi
    for r in range(NW):
        total = total + allcnt_ref[r, pl.ds(0, L)]
    blocks = (total + (BLK - 1)) >> 8
    cblocks = jnp.cumsum(blocks)
    start = (cblocks - blocks) * BLK

    @pl.when(jnp.logical_and(c == 0, s == 0))
    def _():
        be0 = zi
        be1 = zi
        for e in range(E):
            ce = jnp.take(cblocks, zi + e)
            be0 = be0 + jnp.minimum(jnp.maximum(lane - ce + 1, 0), 1)
            be1 = be1 + jnp.minimum(jnp.maximum(lane + L - ce + 1, 0), 1)
        nbv = jnp.take(cblocks, zi + (E - 1))
        oh_nb = oh(NB - L)
        sinfo_v[pl.ds(0, L)] = be0
        sinfo_v[pl.ds(L, L)] = oh_nb * nbv + (1 - oh_nb) * be1
        pltpu.sync_copy(sinfo_v, sinfo_hbm)

    for d in range(2):
        ehbm = (e1_hbm, e2_hbm)[d]
        whbm = (wa_hbm, wb_hbm)[d]
        pltpu.sync_copy(ehbm.at[pl.ds(s * RPW, RPW)], evs_ref)
        pltpu.sync_copy(whbm.at[pl.ds(s * RPW, RPW)], wvs_ref)

        prefix_ref[...] = zi
        for r in range(NW):
            @pl.when(r < s + d * NS)
            def _():
                prefix_ref[...] = prefix_ref[...] + allcnt_ref[r, pl.ds(0, L)]

        rcur = start + prefix_ref[...]
        for ch in range(RPW // L):
            ev = evs_ref[pl.ds(ch * L, L)]
            base = jnp.take(rcur, ev)
            rank = zi
            hist = zi
            for e in range(E):
                mi = 1 - jnp.minimum(jnp.abs(ev - e), 1)
                pc = jnp.cumsum(mi)
                rank = rank + mi * (pc - mi)
                hist = hist + oh(e) * jnp.take(pc, full_last)
            posv = base + rank
            posv = jnp.minimum(jnp.maximum(posv, 0), P - 1)
            posbuf[pl.ds(ch * L, L)] = posv
            tokbuf[pl.ds(ch * L, L)] = (s * RPW + ch * L) + lane
            rcur = rcur + hist
        pltpu.sync_copy(tokbuf, tok_sh.at[posbuf])
        pltpu.sync_copy(wvs_ref, w_sh.at[posbuf])

        @pl.when(c == 0)
        def _():
            pltpu.sync_copy(posbuf, pos_hbm.at[pl.ds(d * T + s * RPW, RPW)])

    plsc.subcore_barrier()
    j = c * NS + s
    pltpu.sync_copy(tok_sh.at[pl.ds(j * SLICE, SLICE)], outbuf_i)
    pltpu.sync_copy(outbuf_i, tok_hbm.at[pl.ds(j * SLICE, SLICE)])
    pltpu.sync_copy(w_sh.at[pl.ds(j * SLICE, SLICE)], outbuf_f)
    pltpu.sync_copy(outbuf_f, w_hbm.at[pl.ds(j * SLICE, SLICE)])


def _route_jnp(e1, e2, wa, wb):
    e_all = jnp.concatenate([e1, e2])
    w_all = jnp.concatenate([wa, wb])
    t_all = jnp.concatenate([jnp.arange(T, dtype=jnp.int32)] * 2)
    onehot = (e_all[:, None] == jnp.arange(E)[None, :]).astype(jnp.int32)
    cnt = jnp.sum(onehot, axis=0)
    blocks = (cnt + BLK - 1) // BLK
    cblocks = jnp.cumsum(blocks)
    nb = cblocks[-1]
    start_blk = jnp.concatenate([jnp.zeros((1,), jnp.int32), cblocks[:-1]])
    rank = jnp.cumsum(onehot, axis=0) - onehot
    pos = start_blk[e_all] * BLK + jnp.take_along_axis(rank, e_all[:, None], 1)[:, 0]
    sorted_token = jnp.zeros((P,), jnp.int32).at[pos].set(t_all)
    sorted_w = jnp.zeros((P,), jnp.float32).at[pos].set(w_all)
    be = jnp.searchsorted(cblocks, jnp.arange(NB, dtype=jnp.int32), side="right")
    be = jnp.clip(be, 0, E - 1).astype(jnp.int32)
    sinfo = jnp.concatenate([be, jnp.zeros((L - E,), jnp.int32),
                             nb[None].astype(jnp.int32),
                             jnp.zeros((L - E - 1,), jnp.int32)])
    sinfo = sinfo.at[0:NB].set(be).at[NB].set(nb)
    return sorted_token, sorted_w, pos, sinfo


def _route(e1, e2, wa, wb):
    mesh = plsc.VectorSubcoreMesh(core_axis_name="c", subcore_axis_name="s")
    f = pl.kernel(
        _route_body,
        out_type=[
            jax.ShapeDtypeStruct((P,), jnp.int32),
            jax.ShapeDtypeStruct((P,), jnp.float32),
            jax.ShapeDtypeStruct((T * K,), jnp.int32),
            jax.ShapeDtypeStruct((2 * L,), jnp.int32),
        ],
        mesh=mesh,
        compiler_params=pltpu.CompilerParams(needs_layout_passes=False),
        scratch_types=[
            pltpu.VMEM((RPW,), jnp.int32),
            pltpu.VMEM((RPW,), jnp.int32),
            pltpu.VMEM((RPW,), jnp.int32),
            pltpu.VMEM((RPW,), jnp.float32),
            pltpu.VMEM_SHARED((NW, L), jnp.int32),
            pltpu.VMEM_SHARED((P,), jnp.int32),
            pltpu.VMEM_SHARED((P,), jnp.float32),
            pltpu.VMEM((NW, L), jnp.int32),
            pltpu.VMEM((1, L), jnp.int32),
            pltpu.VMEM((L,), jnp.int32),
            pltpu.VMEM((RPW,), jnp.int32),
            pltpu.VMEM((RPW,), jnp.int32),
            pltpu.VMEM((SLICE,), jnp.int32),
            pltpu.VMEM((SLICE,), jnp.float32),
            pltpu.VMEM((2 * L,), jnp.int32),
        ],
    )
    return f(e1, e2, wa, wb)


_G_CHUNK = 16
_G_NBUF = 6
_G_N = P // NW // _G_CHUNK

def _gather_body(tok_hbm, x_hbm, xs_hbm, idx_v, *refs):
    rows = list(refs[:_G_NBUF])
    sg = list(refs[_G_NBUF:2 * _G_NBUF])
    ss = list(refs[2 * _G_NBUF:3 * _G_NBUF])
    wid = lax.axis_index("s") * NC + lax.axis_index("c")
    base = wid * (P // NW)
    pltpu.sync_copy(tok_hbm.at[pl.ds(base, P // NW)], idx_v)
    for q in range(P // NW // 16):
        v = idx_v[pl.ds(q * 16, 16)]
        idx_v[pl.ds(q * 16, 16)] = jnp.minimum(jnp.maximum(v, 0), T - 1)

    def g_desc(k):
        b = k % _G_NBUF
        return pltpu.make_async_copy(
            x_hbm.at[idx_v.at[pl.ds(k * _G_CHUNK, _G_CHUNK)]], rows[b], sg[b])

    def s_desc(k):
        b = k % _G_NBUF
        return pltpu.make_async_copy(
            rows[b], xs_hbm.at[pl.ds(base + k * _G_CHUNK, _G_CHUNK)], ss[b])

    for k in range(_G_NBUF):
        g_desc(k).start()
    for k in range(_G_N):
        g_desc(k).wait()
        if k + 1 < _G_N and k + 1 >= _G_NBUF:
            s_desc(k + 1 - _G_NBUF).wait()
            g_desc(k + 1).start()
        s_desc(k).start()
    for k in range(_G_N - _G_NBUF, _G_N):
        s_desc(k).wait()


def _gather_rows(inputs, tok):
    mesh = plsc.VectorSubcoreMesh(core_axis_name="c", subcore_axis_name="s")
    f = pl.kernel(
        _gather_body,
        out_type=jax.ShapeDtypeStruct((P, D), jnp.float32),
        mesh=mesh,
        scratch_types=(
            [pltpu.VMEM((P // NW,), jnp.int32)]
            + [pltpu.VMEM((_G_CHUNK, D), jnp.float32)] * _G_NBUF
            + [pltpu.SemaphoreType.DMA] * (2 * _G_NBUF)
        ),
    )
    return f(tok, inputs)


def _ffn_body(s_ref, xs_ref, w_ref, wt_ref, y_ref):
    i = pl.program_id(0)
    j = pl.program_id(1)
    nb = s_ref[NB]

    @pl.when(i < nb)
    def _():
        x = xs_ref[...].astype(jnp.bfloat16)
        w0 = w_ref[0, 0].astype(jnp.bfloat16)
        w1 = w_ref[0, 1].astype(jnp.bfloat16)
        w2 = w_ref[0, 2].astype(jnp.bfloat16)
        a = jax.lax.dot_general(x, w0, (((1,), (1,)), ((), ())),
                                preferred_element_type=jnp.float32)
        b = jax.lax.dot_general(x, w2, (((1,), (1,)), ((), ())),
                                preferred_element_type=jnp.float32)
        h = a * jax.lax.logistic(a) * b
        part = jax.lax.dot_general(h.astype(jnp.bfloat16), w1,
                                   (((1,), (0,)), ((), ())),
                                   preferred_element_type=jnp.float32)

        @pl.when(j == 0)
        def _():
            y_ref[...] = part

        @pl.when(j > 0)
        def _():
            y_ref[...] = y_ref[...] + part

        @pl.when(j == NSPLIT - 1)
        def _():
            y_ref[...] = y_ref[...] * wt_ref[0, 0, :][:, None]


def _ffn(sinfo, xs, expert_ws, sorted_w):
    wt3 = sorted_w.reshape(NB, 1, BLK)
    grid_spec = pltpu.PrefetchScalarGridSpec(
        num_scalar_prefetch=1,
        grid=(NB, NSPLIT),
        in_specs=[
            pl.BlockSpec((BLK, D), lambda i, j, s: (i, 0)),
            pl.BlockSpec((1, 3, DFF_C, D),
                         lambda i, j, s: (jnp.clip(s[jnp.minimum(i, s[NB] - 1)], 0, E - 1), 0, j, 0)),
            pl.BlockSpec((1, 1, BLK), lambda i, j, s: (i, 0, 0)),
        ],
        out_specs=pl.BlockSpec((BLK, D), lambda i, j, s: (i, 0)),
    )
    return pl.pallas_call(
        _ffn_body,
        grid_spec=grid_spec,
        out_shape=jax.ShapeDtypeStruct((P, D), jnp.float32),
        compiler_params=pltpu.CompilerParams(
            dimension_semantics=("arbitrary", "arbitrary"),
        ),
        interpret=_INTERPRET,
    )(sinfo, xs, expert_ws, wt3)


_C_CHUNK = 8
_C_NBUF = 4
_C_N = T // NW // _C_CHUNK
_TPW = T // NW

def _combine_body(pos_hbm, yw_hbm, out_hbm, i0_v, i1_v, *refs):
    r0 = list(refs[:_C_NBUF])
    r1 = list(refs[_C_NBUF:2 * _C_NBUF])
    sg = list(refs[2 * _C_NBUF:3 * _C_NBUF])
    ss = list(refs[3 * _C_NBUF:4 * _C_NBUF])
    wid = lax.axis_index("s") * NC + lax.axis_index("c")
    base = wid * _TPW
    pltpu.sync_copy(pos_hbm.at[pl.ds(base, _TPW)], i0_v)
    pltpu.sync_copy(pos_hbm.at[pl.ds(T + base, _TPW)], i1_v)

    def g_descs(k):
        b = k % _C_NBUF
        sl = pl.ds(k * _C_CHUNK, _C_CHUNK)
        return (pltpu.make_async_copy(yw_hbm.at[i0_v.at[sl]], r0[b], sg[b]),
                pltpu.make_async_copy(yw_hbm.at[i1_v.at[sl]], r1[b], sg[b]))

    def s_desc(k):
        b = k % _C_NBUF
        return pltpu.make_async_copy(
            r0[b], out_hbm.at[pl.ds(base + k * _C_CHUNK, _C_CHUNK)], ss[b])

    for k in range(_C_NBUF):
        d0, d1 = g_descs(k)
        d0.start()
        d1.start()
    for k in range(_C_N):
        b = k % _C_NBUF
        d0, d1 = g_descs(k)
        d0.wait()
        d1.wait()
        if k + 1 < _C_N and k + 1 >= _C_NBUF:
            s_desc(k + 1 - _C_NBUF).wait()
            n0, n1 = g_descs(k + 1)
            n0.start()
            n1.start()

        def add_body(r, carry):
            for jj in range(D // L):
                csl = pl.ds(jj * L, L)
                r0[b][r, csl] = r0[b][r, csl] + r1[b][r, csl]
            return carry
        lax.fori_loop(0, _C_CHUNK, add_body, 0)
        s_desc(k).start()
    for k in range(_C_N - _C_NBUF, _C_N):
        s_desc(k).wait()


def _combine(yw, pos):
    mesh = plsc.VectorSubcoreMesh(core_axis_name="c", subcore_axis_name="s")
    f = pl.kernel(
        _combine_body,
        out_type=jax.ShapeDtypeStruct((T, D), jnp.float32),
        mesh=mesh,
        scratch_types=(
            [pltpu.VMEM((_TPW,), jnp.int32), pltpu.VMEM((_TPW,), jnp.int32)]
            + [pltpu.VMEM((_C_CHUNK, D), jnp.float32)] * (2 * _C_NBUF)
            + [pltpu.SemaphoreType.DMA] * (2 * _C_NBUF)
        ),
    )
    return f(pos, yw)


def kernel(inputs, gate_w, expert_ws):
    e1, e2, wa, wb = _gate(inputs, gate_w)
    sorted_token, sorted_w, pos, sinfo = _route(e1, e2, wa, wb)
    xs = _gather_rows(inputs, sorted_token)
    yw = _ffn(sinfo, xs, expert_ws, sorted_w)
    return _combine(yw, pos)


# speedup vs baseline: 1.0199x; 1.0199x over previous
import functools

import jax
import jax.numpy as jnp
from jax import lax
from jax.experimental import pallas as pl
from jax.experimental.pallas import tpu as pltpu
from jax.experimental.pallas import tpu_sc as plsc

E = 8
K = 2
T = 2048
D = 1024
DFF = 2816

BLK = 256
NB = (T * K) // BLK + E
P = NB * BLK
NSPLIT = 2
DFF_C = DFF // NSPLIT

NC = 2
NS = 16
NW = NC * NS
L = 16

_INTERPRET = False


def _gate_body(x_ref, gw_ref, e1_ref, e2_ref, w1_ref, w2_ref):
    x = x_ref[...]
    gl = jax.lax.dot_general(x, gw_ref[...], (((1,), (1,)), ((), ())))
    iota = jax.lax.broadcasted_iota(jnp.int32, gl.shape, 1)
    m1 = jnp.max(gl, axis=1, keepdims=True)
    a1 = jnp.min(jnp.where(gl == m1, iota, E), axis=1, keepdims=True)
    masked = jnp.where(iota == a1, -jnp.inf, gl)
    m2 = jnp.max(masked, axis=1, keepdims=True)
    a2 = jnp.min(jnp.where(masked == m2, iota, E), axis=1, keepdims=True)
    p1 = 1.0 / (1.0 + jnp.exp(m2 - m1))
    e1_ref[...] = a1[:, 0]
    e2_ref[...] = a2[:, 0]
    w1_ref[...] = p1[:, 0]
    w2_ref[...] = 1.0 - p1[:, 0]


def _gate(inputs, gate_w):
    nblk = T // BLK
    return pl.pallas_call(
        _gate_body,
        grid=(nblk,),
        in_specs=[
            pl.BlockSpec((BLK, D), lambda i: (i, 0)),
            pl.BlockSpec((E, D), lambda i: (0, 0)),
        ],
        out_specs=[
            pl.BlockSpec((BLK,), lambda i: (i,)),
            pl.BlockSpec((BLK,), lambda i: (i,)),
            pl.BlockSpec((BLK,), lambda i: (i,)),
            pl.BlockSpec((BLK,), lambda i: (i,)),
        ],
        out_shape=[
            jax.ShapeDtypeStruct((T,), jnp.int32),
            jax.ShapeDtypeStruct((T,), jnp.int32),
            jax.ShapeDtypeStruct((T,), jnp.float32),
            jax.ShapeDtypeStruct((T,), jnp.float32),
        ],
        interpret=_INTERPRET,
    )(inputs, gate_w)


RPW = (T * K) // NW
SLICE = P // NW
_G_CHUNK = 16
_G_NBUF = 6
_G_N = SLICE // _G_CHUNK


def _route_body(e1_hbm, e2_hbm, wa_hbm, wb_hbm, x_hbm,
                w_hbm, pos_hbm, sinfo_hbm, xs_hbm,
                eva_ref, evb_ref, evs_ref, wvs_ref,
                cnt_sh, tok_sh, w_sh,
                allcnt_ref, cntbuf_ref, prefix_ref,
                tokbuf, posbuf, outbuf_f, sinfo_v, idx_v, *grefs):
    rows = list(grefs[:_G_NBUF])
    sg = list(grefs[_G_NBUF:2 * _G_NBUF])
    ss = list(grefs[2 * _G_NBUF:3 * _G_NBUF])
    c = lax.axis_index("c")
    s = lax.axis_index("s")
    lane = lax.broadcasted_iota(jnp.int32, (L,), 0)
    zi = lane * 0
    full_last = zi + (L - 1)

    def oh(e):
        return 1 - jnp.minimum(jnp.abs(lane - e), 1)

    pltpu.sync_copy(e1_hbm.at[pl.ds(s * RPW, RPW)], eva_ref)
    pltpu.sync_copy(e2_hbm.at[pl.ds(s * RPW, RPW)], evb_ref)

    def count_range(ev_ref):
        cnt = zi
        for ch in range(RPW // L):
            ev = ev_ref[pl.ds(ch * L, L)]
            for e in range(E):
                mi = 1 - jnp.minimum(jnp.abs(ev - e), 1)
                pc = jnp.cumsum(mi)
                cnt = cnt + oh(e) * jnp.take(pc, full_last)
        return cnt

    cntbuf_ref[0, pl.ds(0, L)] = count_range(eva_ref)
    pltpu.sync_copy(cntbuf_ref, cnt_sh.at[pl.ds(s, 1)])
    cntbuf_ref[0, pl.ds(0, L)] = count_range(evb_ref)
    pltpu.sync_copy(cntbuf_ref, cnt_sh.at[pl.ds(s + NS, 1)])
    plsc.subcore_barrier()
    pltpu.sync_copy(cnt_sh, allcnt_ref)

    total = zi
    for r in range(NW):
        total = total + allcnt_ref[r, pl.ds(0, L)]
    blocks = (total + (BLK - 1)) >> 8
    cblocks = jnp.cumsum(blocks)
    start = (cblocks - blocks) * BLK

    @pl.when(jnp.logical_and(c == 0, s == 0))
    def _():
        be0 = zi
        be1 = zi
        for e in range(E):
            ce = jnp.take(cblocks, zi + e)
            be0 = be0 + jnp.minimum(jnp.maximum(lane - ce + 1, 0), 1)
            be1 = be1 + jnp.minimum(jnp.maximum(lane + L - ce + 1, 0), 1)
        nbv = jnp.take(cblocks, zi + (E - 1))
        oh_nb = oh(NB - L)
        sinfo_v[pl.ds(0, L)] = be0
        sinfo_v[pl.ds(L, L)] = oh_nb * nbv + (1 - oh_nb) * be1
        pltpu.sync_copy(sinfo_v, sinfo_hbm)

    for d in range(2):
        ehbm = (e1_hbm, e2_hbm)[d]
        whbm = (wa_hbm, wb_hbm)[d]
        pltpu.sync_copy(ehbm.at[pl.ds(s * RPW, RPW)], evs_ref)
        pltpu.sync_copy(whbm.at[pl.ds(s * RPW, RPW)], wvs_ref)

        prefix_ref[...] = zi
        for r in range(NW):
            @pl.when(r < s + d * NS)
            def _():
                prefix_ref[...] = prefix_ref[...] + allcnt_ref[r, pl.ds(0, L)]

        rcur = start + prefix_ref[...]
        for ch in range(RPW // L):
            ev = evs_ref[pl.ds(ch * L, L)]
            base = jnp.take(rcur, ev)
            rank = zi
            hist = zi
            for e in range(E):
                mi = 1 - jnp.minimum(jnp.abs(ev - e), 1)
                pc = jnp.cumsum(mi)
                rank = rank + mi * (pc - mi)
                hist = hist + oh(e) * jnp.take(pc, full_last)
            posv = base + rank
            posv = jnp.minimum(jnp.maximum(posv, 0), P - 1)
            posbuf[pl.ds(ch * L, L)] = posv
            tokbuf[pl.ds(ch * L, L)] = (s * RPW + ch * L) + lane
            rcur = rcur + hist
        pltpu.sync_copy(tokbuf, tok_sh.at[posbuf])
        pltpu.sync_copy(wvs_ref, w_sh.at[posbuf])

        @pl.when(c == 0)
        def _():
            pltpu.sync_copy(posbuf, pos_hbm.at[pl.ds(d * T + s * RPW, RPW)])

    plsc.subcore_barrier()
    j = c * NS + s
    pltpu.sync_copy(w_sh.at[pl.ds(j * SLICE, SLICE)], outbuf_f)
    pltpu.sync_copy(outbuf_f, w_hbm.at[pl.ds(j * SLICE, SLICE)])

    base = j * SLICE
    pltpu.sync_copy(tok_sh.at[pl.ds(base, SLICE)], idx_v)
    for q in range(SLICE // L):
        v = idx_v[pl.ds(q * L, L)]
        idx_v[pl.ds(q * L, L)] = jnp.minimum(jnp.maximum(v, 0), T - 1)

    def g_desc(k):
        b = k % _G_NBUF
        return pltpu.make_async_copy(
            x_hbm.at[idx_v.at[pl.ds(k * _G_CHUNK, _G_CHUNK)]], rows[b], sg[b])

    def s_desc(k):
        b = k % _G_NBUF
        return pltpu.make_async_copy(
            rows[b], xs_hbm.at[pl.ds(base + k * _G_CHUNK, _G_CHUNK)], ss[b])

    for k in range(_G_NBUF):
        g_desc(k).start()
    for k in range(_G_N):
        g_desc(k).wait()
        if k + 1 < _G_N and k + 1 >= _G_NBUF:
            s_desc(k + 1 - _G_NBUF).wait()
            g_desc(k + 1).start()
        s_desc(k).start()
    for k in range(_G_N - _G_NBUF, _G_N):
        s_desc(k).wait()


def _route_jnp(e1, e2, wa, wb):
    e_all = jnp.concatenate([e1, e2])
    w_all = jnp.concatenate([wa, wb])
    t_all = jnp.concatenate([jnp.arange(T, dtype=jnp.int32)] * 2)
    onehot = (e_all[:, None] == jnp.arange(E)[None, :]).astype(jnp.int32)
    cnt = jnp.sum(onehot, axis=0)
    blocks = (cnt + BLK - 1) // BLK
    cblocks = jnp.cumsum(blocks)
    nb = cblocks[-1]
    start_blk = jnp.concatenate([jnp.zeros((1,), jnp.int32), cblocks[:-1]])
    rank = jnp.cumsum(onehot, axis=0) - onehot
    pos = start_blk[e_all] * BLK + jnp.take_along_axis(rank, e_all[:, None], 1)[:, 0]
    sorted_token = jnp.zeros((P,), jnp.int32).at[pos].set(t_all)
    sorted_w = jnp.zeros((P,), jnp.float32).at[pos].set(w_all)
    be = jnp.searchsorted(cblocks, jnp.arange(NB, dtype=jnp.int32), side="right")
    be = jnp.clip(be, 0, E - 1).astype(jnp.int32)
    sinfo = jnp.concatenate([be, jnp.zeros((L - E,), jnp.int32),
                             nb[None].astype(jnp.int32),
                             jnp.zeros((L - E - 1,), jnp.int32)])
    sinfo = sinfo.at[0:NB].set(be).at[NB].set(nb)
    return sorted_token, sorted_w, pos, sinfo


def _route(e1, e2, wa, wb, inputs):
    mesh = plsc.VectorSubcoreMesh(core_axis_name="c", subcore_axis_name="s")
    f = pl.kernel(
        _route_body,
        out_type=[
            jax.ShapeDtypeStruct((P,), jnp.float32),
            jax.ShapeDtypeStruct((T * K,), jnp.int32),
            jax.ShapeDtypeStruct((2 * L,), jnp.int32),
            jax.ShapeDtypeStruct((P, D), jnp.float32),
        ],
        mesh=mesh,
        compiler_params=pltpu.CompilerParams(needs_layout_passes=False),
        scratch_types=(
            [
                pltpu.VMEM((RPW,), jnp.int32),
                pltpu.VMEM((RPW,), jnp.int32),
                pltpu.VMEM((RPW,), jnp.int32),
                pltpu.VMEM((RPW,), jnp.float32),
                pltpu.VMEM_SHARED((NW, L), jnp.int32),
                pltpu.VMEM_SHARED((P,), jnp.int32),
                pltpu.VMEM_SHARED((P,), jnp.float32),
                pltpu.VMEM((NW, L), jnp.int32),
                pltpu.VMEM((1, L), jnp.int32),
                pltpu.VMEM((L,), jnp.int32),
                pltpu.VMEM((RPW,), jnp.int32),
                pltpu.VMEM((RPW,), jnp.int32),
                pltpu.VMEM((SLICE,), jnp.float32),
                pltpu.VMEM((2 * L,), jnp.int32),
                pltpu.VMEM((SLICE,), jnp.int32),
            ]
            + [pltpu.VMEM((_G_CHUNK, D), jnp.float32)] * _G_NBUF
            + [pltpu.SemaphoreType.DMA] * (2 * _G_NBUF)
        ),
    )
    return f(e1, e2, wa, wb, inputs)


def _ffn_body(s_ref, xs_ref, w_ref, wt_ref, y_ref):
    i = pl.program_id(0)
    j = pl.program_id(1)
    nb = s_ref[NB]

    @pl.when(i < nb)
    def _():
        x = xs_ref[...].astype(jnp.bfloat16)
        w0 = w_ref[0, 0].astype(jnp.bfloat16)
        w1 = w_ref[0, 1].astype(jnp.bfloat16)
        w2 = w_ref[0, 2].astype(jnp.bfloat16)
        a = jax.lax.dot_general(x, w0, (((1,), (1,)), ((), ())),
                                preferred_element_type=jnp.float32)
        b = jax.lax.dot_general(x, w2, (((1,), (1,)), ((), ())),
                                preferred_element_type=jnp.float32)
        h = a * jax.lax.logistic(a) * b
        part = jax.lax.dot_general(h.astype(jnp.bfloat16), w1,
                                   (((1,), (0,)), ((), ())),
                                   preferred_element_type=jnp.float32)

        @pl.when(j == 0)
        def _():
            y_ref[...] = part

        @pl.when(j > 0)
        def _():
            y_ref[...] = y_ref[...] + part

        @pl.when(j == NSPLIT - 1)
        def _():
            y_ref[...] = y_ref[...] * wt_ref[0, 0, :][:, None]


def _ffn(sinfo, xs, expert_ws, sorted_w):
    wt3 = sorted_w.reshape(NB, 1, BLK)
    grid_spec = pltpu.PrefetchScalarGridSpec(
        num_scalar_prefetch=1,
        grid=(NB, NSPLIT),
        in_specs=[
            pl.BlockSpec((BLK, D), lambda i, j, s: (i, 0)),
            pl.BlockSpec((1, 3, DFF_C, D),
                         lambda i, j, s: (jnp.clip(s[jnp.minimum(i, s[NB] - 1)], 0, E - 1), 0, j, 0)),
            pl.BlockSpec((1, 1, BLK), lambda i, j, s: (i, 0, 0)),
        ],
        out_specs=pl.BlockSpec((BLK, D), lambda i, j, s: (i, 0)),
    )
    return pl.pallas_call(
        _ffn_body,
        grid_spec=grid_spec,
        out_shape=jax.ShapeDtypeStruct((P, D), jnp.float32),
        compiler_params=pltpu.CompilerParams(
            dimension_semantics=("arbitrary", "arbitrary"),
        ),
        interpret=_INTERPRET,
    )(sinfo, xs, expert_ws, wt3)


_C_CHUNK = 8
_C_NBUF = 4
_C_N = T // NW // _C_CHUNK
_TPW = T // NW

def _combine_body(pos_hbm, yw_hbm, out_hbm, i0_v, i1_v, *refs):
    r0 = list(refs[:_C_NBUF])
    r1 = list(refs[_C_NBUF:2 * _C_NBUF])
    sg = list(refs[2 * _C_NBUF:3 * _C_NBUF])
    ss = list(refs[3 * _C_NBUF:4 * _C_NBUF])
    wid = lax.axis_index("s") * NC + lax.axis_index("c")
    base = wid * _TPW
    pltpu.sync_copy(pos_hbm.at[pl.ds(base, _TPW)], i0_v)
    pltpu.sync_copy(pos_hbm.at[pl.ds(T + base, _TPW)], i1_v)

    def g_descs(k):
        b = k % _C_NBUF
        sl = pl.ds(k * _C_CHUNK, _C_CHUNK)
        return (pltpu.make_async_copy(yw_hbm.at[i0_v.at[sl]], r0[b], sg[b]),
                pltpu.make_async_copy(yw_hbm.at[i1_v.at[sl]], r1[b], sg[b]))

    def s_desc(k):
        b = k % _C_NBUF
        return pltpu.make_async_copy(
            r0[b], out_hbm.at[pl.ds(base + k * _C_CHUNK, _C_CHUNK)], ss[b])

    for k in range(_C_NBUF):
        d0, d1 = g_descs(k)
        d0.start()
        d1.start()
    for k in range(_C_N):
        b = k % _C_NBUF
        d0, d1 = g_descs(k)
        d0.wait()
        d1.wait()
        if k + 1 < _C_N and k + 1 >= _C_NBUF:
            s_desc(k + 1 - _C_NBUF).wait()
            n0, n1 = g_descs(k + 1)
            n0.start()
            n1.start()

        def add_body(r, carry):
            for jj in range(D // L):
                csl = pl.ds(jj * L, L)
                r0[b][r, csl] = r0[b][r, csl] + r1[b][r, csl]
            return carry
        lax.fori_loop(0, _C_CHUNK, add_body, 0)
        s_desc(k).start()
    for k in range(_C_N - _C_NBUF, _C_N):
        s_desc(k).wait()


def _combine(yw, pos):
    mesh = plsc.VectorSubcoreMesh(core_axis_name="c", subcore_axis_name="s")
    f = pl.kernel(
        _combine_body,
        out_type=jax.ShapeDtypeStruct((T, D), jnp.float32),
        mesh=mesh,
        scratch_types=(
            [pltpu.VMEM((_TPW,), jnp.int32), pltpu.VMEM((_TPW,), jnp.int32)]
            + [pltpu.VMEM((_C_CHUNK, D), jnp.float32)] * (2 * _C_NBUF)
            + [pltpu.SemaphoreType.DMA] * (2 * _C_NBUF)
        ),
    )
    return f(pos, yw)


def kernel(inputs, gate_w, expert_ws):
    e1, e2, wa, wb = _gate(inputs, gate_w)
    sorted_w, pos, sinfo, xs = _route(e1, e2, wa, wb, inputs)
    yw = _ffn(sinfo, xs, expert_ws, sorted_w)
    return _combine(yw, pos)


# speedup vs baseline: 1.2250x; 1.2011x over previous
import functools

import jax
import jax.numpy as jnp
from jax import lax
from jax.experimental import pallas as pl
from jax.experimental.pallas import tpu as pltpu
from jax.experimental.pallas import tpu_sc as plsc

E = 8
K = 2
T = 2048
D = 1024
DFF = 2816

BLK = 256
NB = (T * K) // BLK + E
P = NB * BLK
NSPLIT = 2
DFF_C = DFF // NSPLIT

NC = 2
NS = 16
NW = NC * NS
L = 16

_INTERPRET = False


def _gate_body(x_ref, gw_ref, e1_ref, e2_ref, w1_ref, w2_ref):
    x = x_ref[...]
    gl = jax.lax.dot_general(x, gw_ref[...], (((1,), (1,)), ((), ())))
    iota = jax.lax.broadcasted_iota(jnp.int32, gl.shape, 1)
    m1 = jnp.max(gl, axis=1, keepdims=True)
    a1 = jnp.min(jnp.where(gl == m1, iota, E), axis=1, keepdims=True)
    masked = jnp.where(iota == a1, -jnp.inf, gl)
    m2 = jnp.max(masked, axis=1, keepdims=True)
    a2 = jnp.min(jnp.where(masked == m2, iota, E), axis=1, keepdims=True)
    p1 = 1.0 / (1.0 + jnp.exp(m2 - m1))
    e1_ref[...] = a1[:, 0]
    e2_ref[...] = a2[:, 0]
    w1_ref[...] = p1[:, 0]
    w2_ref[...] = 1.0 - p1[:, 0]


def _gate(inputs, gate_w):
    nblk = T // BLK
    return pl.pallas_call(
        _gate_body,
        grid=(nblk,),
        in_specs=[
            pl.BlockSpec((BLK, D), lambda i: (i, 0)),
            pl.BlockSpec((E, D), lambda i: (0, 0)),
        ],
        out_specs=[
            pl.BlockSpec((BLK,), lambda i: (i,)),
            pl.BlockSpec((BLK,), lambda i: (i,)),
            pl.BlockSpec((BLK,), lambda i: (i,)),
            pl.BlockSpec((BLK,), lambda i: (i,)),
        ],
        out_shape=[
            jax.ShapeDtypeStruct((T,), jnp.int32),
            jax.ShapeDtypeStruct((T,), jnp.int32),
            jax.ShapeDtypeStruct((T,), jnp.float32),
            jax.ShapeDtypeStruct((T,), jnp.float32),
        ],
        interpret=_INTERPRET,
    )(inputs, gate_w)


RPW = (T * K) // NW
SLICE = P // NW
_G_CHUNK = 16
_G_NBUF = 6
_G_N = SLICE // _G_CHUNK


def _route_body(e1_hbm, e2_hbm, wa_hbm, wb_hbm, x_hbm,
                w_hbm, pos_hbm, sinfo_hbm, xs_hbm,
                eva_ref, evb_ref, evs_ref, wvs_ref,
                cnt_sh, tok_sh, w_sh,
                allcnt_ref, cntbuf_ref, prefix_ref,
                tokbuf, posbuf, outbuf_f, sinfo_v, idx_v, *grefs):
    rows = list(grefs[:_G_NBUF])
    sg = list(grefs[_G_NBUF:2 * _G_NBUF])
    ss = list(grefs[2 * _G_NBUF:3 * _G_NBUF])
    c = lax.axis_index("c")
    s = lax.axis_index("s")
    lane = lax.broadcasted_iota(jnp.int32, (L,), 0)
    zi = lane * 0
    full_last = zi + (L - 1)

    def oh(e):
        return 1 - jnp.minimum(jnp.abs(lane - e), 1)

    pltpu.sync_copy(e1_hbm.at[pl.ds(s * RPW, RPW)], eva_ref)
    pltpu.sync_copy(e2_hbm.at[pl.ds(s * RPW, RPW)], evb_ref)

    def count_range(ev_ref):
        cnt = zi
        for ch in range(RPW // L):
            ev = ev_ref[pl.ds(ch * L, L)]
            for e in range(E):
                mi = 1 - jnp.minimum(jnp.abs(ev - e), 1)
                pc = jnp.cumsum(mi)
                cnt = cnt + oh(e) * jnp.take(pc, full_last)
        return cnt

    cntbuf_ref[0, pl.ds(0, L)] = count_range(eva_ref)
    pltpu.sync_copy(cntbuf_ref, cnt_sh.at[pl.ds(s, 1)])
    cntbuf_ref[0, pl.ds(0, L)] = count_range(evb_ref)
    pltpu.sync_copy(cntbuf_ref, cnt_sh.at[pl.ds(s + NS, 1)])
    plsc.subcore_barrier()
    pltpu.sync_copy(cnt_sh, allcnt_ref)

    total = zi
    for r in range(NW):
        total = total + allcnt_ref[r, pl.ds(0, L)]
    blocks = (total + (BLK - 1)) >> 8
    cblocks = jnp.cumsum(blocks)
    start = (cblocks - blocks) * BLK

    @pl.when(jnp.logical_and(c == 0, s == 0))
    def _():
        be0 = zi
        be1 = zi
        for e in range(E):
            ce = jnp.take(cblocks, zi + e)
            be0 = be0 + jnp.minimum(jnp.maximum(lane - ce + 1, 0), 1)
            be1 = be1 + jnp.minimum(jnp.maximum(lane + L - ce + 1, 0), 1)
        nbv = jnp.take(cblocks, zi + (E - 1))
        oh_nb = oh(NB - L)
        sinfo_v[pl.ds(0, L)] = be0
        sinfo_v[pl.ds(L, L)] = oh_nb * nbv + (1 - oh_nb) * be1
        pltpu.sync_copy(sinfo_v, sinfo_hbm)

    for d in range(2):
        ehbm = (e1_hbm, e2_hbm)[d]
        whbm = (wa_hbm, wb_hbm)[d]
        pltpu.sync_copy(ehbm.at[pl.ds(s * RPW, RPW)], evs_ref)
        pltpu.sync_copy(whbm.at[pl.ds(s * RPW, RPW)], wvs_ref)

        prefix_ref[...] = zi
        for r in range(NW):
            @pl.when(r < s + d * NS)
            def _():
                prefix_ref[...] = prefix_ref[...] + allcnt_ref[r, pl.ds(0, L)]

        rcur = start + prefix_ref[...]
        for ch in range(RPW // L):
            ev = evs_ref[pl.ds(ch * L, L)]
            base = jnp.take(rcur, ev)
            rank = zi
            hist = zi
            for e in range(E):
                mi = 1 - jnp.minimum(jnp.abs(ev - e), 1)
                pc = jnp.cumsum(mi)
                rank = rank + mi * (pc - mi)
                hist = hist + oh(e) * jnp.take(pc, full_last)
            posv = base + rank
            posv = jnp.minimum(jnp.maximum(posv, 0), P - 1)
            posbuf[pl.ds(ch * L, L)] = posv
            tokbuf[pl.ds(ch * L, L)] = (s * RPW + ch * L) + lane
            rcur = rcur + hist
        pltpu.sync_copy(tokbuf, tok_sh.at[posbuf])
        pltpu.sync_copy(wvs_ref, w_sh.at[posbuf])

        @pl.when(c == 0)
        def _():
            pltpu.sync_copy(posbuf, pos_hbm.at[pl.ds(d * T + s * RPW, RPW)])

    plsc.subcore_barrier()
    j = c * NS + s
    pltpu.sync_copy(w_sh.at[pl.ds(j * SLICE, SLICE)], outbuf_f)
    pltpu.sync_copy(outbuf_f, w_hbm.at[pl.ds(j * SLICE, SLICE)])

    base = j * SLICE
    pltpu.sync_copy(tok_sh.at[pl.ds(base, SLICE)], idx_v)
    for q in range(SLICE // L):
        v = idx_v[pl.ds(q * L, L)]
        idx_v[pl.ds(q * L, L)] = jnp.minimum(jnp.maximum(v, 0), T - 1)

    def g_desc(k):
        b = k % _G_NBUF
        return pltpu.make_async_copy(
            x_hbm.at[idx_v.at[pl.ds(k * _G_CHUNK, _G_CHUNK)]], rows[b], sg[b])

    def s_desc(k):
        b = k % _G_NBUF
        return pltpu.make_async_copy(
            rows[b], xs_hbm.at[pl.ds(base + k * _G_CHUNK, _G_CHUNK)], ss[b])

    for k in range(_G_NBUF):
        g_desc(k).start()
    for k in range(_G_N):
        g_desc(k).wait()
        if k + 1 < _G_N and k + 1 >= _G_NBUF:
            s_desc(k + 1 - _G_NBUF).wait()
            g_desc(k + 1).start()
        s_desc(k).start()
    for k in range(_G_N - _G_NBUF, _G_N):
        s_desc(k).wait()


def _route_jnp(e1, e2, wa, wb):
    e_all = jnp.concatenate([e1, e2])
    w_all = jnp.concatenate([wa, wb])
    t_all = jnp.concatenate([jnp.arange(T, dtype=jnp.int32)] * 2)
    onehot = (e_all[:, None] == jnp.arange(E)[None, :]).astype(jnp.int32)
    cnt = jnp.sum(onehot, axis=0)
    blocks = (cnt + BLK - 1) // BLK
    cblocks = jnp.cumsum(blocks)
    nb = cblocks[-1]
    start_blk = jnp.concatenate([jnp.zeros((1,), jnp.int32), cblocks[:-1]])
    rank = jnp.cumsum(onehot, axis=0) - onehot
    pos = start_blk[e_all] * BLK + jnp.take_along_axis(rank, e_all[:, None], 1)[:, 0]
    sorted_token = jnp.zeros((P,), jnp.int32).at[pos].set(t_all)
    sorted_w = jnp.zeros((P,), jnp.float32).at[pos].set(w_all)
    be = jnp.searchsorted(cblocks, jnp.arange(NB, dtype=jnp.int32), side="right")
    be = jnp.clip(be, 0, E - 1).astype(jnp.int32)
    sinfo = jnp.concatenate([be, jnp.zeros((L - E,), jnp.int32),
                             nb[None].astype(jnp.int32),
                             jnp.zeros((L - E - 1,), jnp.int32)])
    sinfo = sinfo.at[0:NB].set(be).at[NB].set(nb)
    return sorted_token, sorted_w, pos, sinfo


def _route(e1, e2, wa, wb, inputs):
    mesh = plsc.VectorSubcoreMesh(core_axis_name="c", subcore_axis_name="s")
    f = pl.kernel(
        _route_body,
        out_type=[
            jax.ShapeDtypeStruct((P,), jnp.float32),
            jax.ShapeDtypeStruct((T * K,), jnp.int32),
            jax.ShapeDtypeStruct((2 * L,), jnp.int32),
            jax.ShapeDtypeStruct((P, D), jnp.float32),
        ],
        mesh=mesh,
        compiler_params=pltpu.CompilerParams(needs_layout_passes=False),
        scratch_types=(
            [
                pltpu.VMEM((RPW,), jnp.int32),
                pltpu.VMEM((RPW,), jnp.int32),
                pltpu.VMEM((RPW,), jnp.int32),
                pltpu.VMEM((RPW,), jnp.float32),
                pltpu.VMEM_SHARED((NW, L), jnp.int32),
                pltpu.VMEM_SHARED((P,), jnp.int32),
                pltpu.VMEM_SHARED((P,), jnp.float32),
                pltpu.VMEM((NW, L), jnp.int32),
                pltpu.VMEM((1, L), jnp.int32),
                pltpu.VMEM((L,), jnp.int32),
                pltpu.VMEM((RPW,), jnp.int32),
                pltpu.VMEM((RPW,), jnp.int32),
                pltpu.VMEM((SLICE,), jnp.float32),
                pltpu.VMEM((2 * L,), jnp.int32),
                pltpu.VMEM((SLICE,), jnp.int32),
            ]
            + [pltpu.VMEM((_G_CHUNK, D), jnp.float32)] * _G_NBUF
            + [pltpu.SemaphoreType.DMA] * (2 * _G_NBUF)
        ),
    )
    return f(e1, e2, wa, wb, inputs)


def _ffn1_body(s_ref, xs_ref, w0_ref, w2_ref, h_ref):
    i = pl.program_id(0)
    nb = s_ref[NB]

    @pl.when(i < nb)
    def _():
        x = xs_ref[...].astype(jnp.bfloat16)
        w0 = w0_ref[0, 0].astype(jnp.bfloat16)
        w2 = w2_ref[0, 0].astype(jnp.bfloat16)
        a = jax.lax.dot_general(x, w0, (((1,), (1,)), ((), ())),
                                preferred_element_type=jnp.float32)
        b = jax.lax.dot_general(x, w2, (((1,), (1,)), ((), ())),
                                preferred_element_type=jnp.float32)
        h_ref[...] = (a * jax.lax.logistic(a) * b).astype(jnp.bfloat16)


def _ffn2_body(s_ref, h_ref, w1_ref, wt_ref, y_ref):
    i = pl.program_id(0)
    nb = s_ref[NB]

    @pl.when(i < nb)
    def _():
        w1 = w1_ref[0, 0].astype(jnp.bfloat16)
        part = jax.lax.dot_general(h_ref[...], w1, (((1,), (0,)), ((), ())),
                                   preferred_element_type=jnp.float32)
        y_ref[...] = part * wt_ref[0, 0, :][:, None]


def _ffn(sinfo, xs, expert_ws, sorted_w):
    wt3 = sorted_w.reshape(NB, 1, BLK)

    def emap(slot):
        def f(i, s):
            return (jnp.clip(s[jnp.minimum(i, s[NB] - 1)], 0, E - 1), slot, 0, 0)
        return f

    h = pl.pallas_call(
        _ffn1_body,
        grid_spec=pltpu.PrefetchScalarGridSpec(
            num_scalar_prefetch=1,
            grid=(NB,),
            in_specs=[
                pl.BlockSpec((BLK, D), lambda i, s: (i, 0)),
                pl.BlockSpec((1, 1, DFF, D), emap(0)),
                pl.BlockSpec((1, 1, DFF, D), emap(2)),
            ],
            out_specs=pl.BlockSpec((BLK, DFF), lambda i, s: (i, 0)),
        ),
        out_shape=jax.ShapeDtypeStruct((P, DFF), jnp.bfloat16),
        compiler_params=pltpu.CompilerParams(
            dimension_semantics=("arbitrary",),
            vmem_limit_bytes=60 * 1024 * 1024,
        ),
        interpret=_INTERPRET,
    )(sinfo, xs, expert_ws, expert_ws)

    return pl.pallas_call(
        _ffn2_body,
        grid_spec=pltpu.PrefetchScalarGridSpec(
            num_scalar_prefetch=1,
            grid=(NB,),
            in_specs=[
                pl.BlockSpec((BLK, DFF), lambda i, s: (i, 0)),
                pl.BlockSpec((1, 1, DFF, D), emap(1)),
                pl.BlockSpec((1, 1, BLK), lambda i, s: (i, 0, 0)),
            ],
            out_specs=pl.BlockSpec((BLK, D), lambda i, s: (i, 0)),
        ),
        out_shape=jax.ShapeDtypeStruct((P, D), jnp.float32),
        compiler_params=pltpu.CompilerParams(
            dimension_semantics=("arbitrary",),
            vmem_limit_bytes=60 * 1024 * 1024,
        ),
        interpret=_INTERPRET,
    )(sinfo, h, expert_ws, wt3)


_C_CHUNK = 8
_C_NBUF = 4
_C_N = T // NW // _C_CHUNK
_TPW = T // NW

def _combine_body(pos_hbm, yw_hbm, out_hbm, i0_v, i1_v, *refs):
    r0 = list(refs[:_C_NBUF])
    r1 = list(refs[_C_NBUF:2 * _C_NBUF])
    sg = list(refs[2 * _C_NBUF:3 * _C_NBUF])
    ss = list(refs[3 * _C_NBUF:4 * _C_NBUF])
    wid = lax.axis_index("s") * NC + lax.axis_index("c")
    base = wid * _TPW
    pltpu.sync_copy(pos_hbm.at[pl.ds(base, _TPW)], i0_v)
    pltpu.sync_copy(pos_hbm.at[pl.ds(T + base, _TPW)], i1_v)

    def g_descs(k):
        b = k % _C_NBUF
        sl = pl.ds(k * _C_CHUNK, _C_CHUNK)
        return (pltpu.make_async_copy(yw_hbm.at[i0_v.at[sl]], r0[b], sg[b]),
                pltpu.make_async_copy(yw_hbm.at[i1_v.at[sl]], r1[b], sg[b]))

    def s_desc(k):
        b = k % _C_NBUF
        return pltpu.make_async_copy(
            r0[b], out_hbm.at[pl.ds(base + k * _C_CHUNK, _C_CHUNK)], ss[b])

    for k in range(_C_NBUF):
        d0, d1 = g_descs(k)
        d0.start()
        d1.start()
    for k in range(_C_N):
        b = k % _C_NBUF
        d0, d1 = g_descs(k)
        d0.wait()
        d1.wait()
        if k + 1 < _C_N and k + 1 >= _C_NBUF:
            s_desc(k + 1 - _C_NBUF).wait()
            n0, n1 = g_descs(k + 1)
            n0.start()
            n1.start()

        def add_body(r, carry):
            for jj in range(D // L):
                csl = pl.ds(jj * L, L)
                r0[b][r, csl] = r0[b][r, csl] + r1[b][r, csl]
            return carry
        lax.fori_loop(0, _C_CHUNK, add_body, 0)
        s_desc(k).start()
    for k in range(_C_N - _C_NBUF, _C_N):
        s_desc(k).wait()


def _combine(yw, pos):
    mesh = plsc.VectorSubcoreMesh(core_axis_name="c", subcore_axis_name="s")
    f = pl.kernel(
        _combine_body,
        out_type=jax.ShapeDtypeStruct((T, D), jnp.float32),
        mesh=mesh,
        scratch_types=(
            [pltpu.VMEM((_TPW,), jnp.int32), pltpu.VMEM((_TPW,), jnp.int32)]
            + [pltpu.VMEM((_C_CHUNK, D), jnp.float32)] * (2 * _C_NBUF)
            + [pltpu.SemaphoreType.DMA] * (2 * _C_NBUF)
        ),
    )
    return f(pos, yw)


def kernel(inputs, gate_w, expert_ws):
    e1, e2, wa, wb = _gate(inputs, gate_w)
    sorted_w, pos, sinfo, xs = _route(e1, e2, wa, wb, inputs)
    yw = _ffn(sinfo, xs, expert_ws, sorted_w)
    return _combine(yw, pos)


# speedup vs baseline: 1.4719x; 1.2016x over previous
import functools

import jax
import jax.numpy as jnp
from jax import lax
from jax.experimental import pallas as pl
from jax.experimental.pallas import tpu as pltpu
from jax.experimental.pallas import tpu_sc as plsc

E = 8
K = 2
T = 2048
D = 1024
DFF = 2816

BLK = 256
NB = (T * K) // BLK + E
P = NB * BLK
NSPLIT = 2
DFF_C = DFF // NSPLIT

NC = 2
NS = 16
NW = NC * NS
L = 16

_INTERPRET = False


def _gate_body(x_ref, gw_ref, e1_ref, e2_ref, w1_ref, w2_ref):
    x = x_ref[...]
    gl = jax.lax.dot_general(x, gw_ref[...], (((1,), (1,)), ((), ())))
    iota = jax.lax.broadcasted_iota(jnp.int32, gl.shape, 1)
    m1 = jnp.max(gl, axis=1, keepdims=True)
    a1 = jnp.min(jnp.where(gl == m1, iota, E), axis=1, keepdims=True)
    masked = jnp.where(iota == a1, -jnp.inf, gl)
    m2 = jnp.max(masked, axis=1, keepdims=True)
    a2 = jnp.min(jnp.where(masked == m2, iota, E), axis=1, keepdims=True)
    p1 = 1.0 / (1.0 + jnp.exp(m2 - m1))
    e1_ref[...] = a1[:, 0]
    e2_ref[...] = a2[:, 0]
    w1_ref[...] = p1[:, 0]
    w2_ref[...] = 1.0 - p1[:, 0]


def _gate(inputs, gate_w):
    nblk = T // BLK
    return pl.pallas_call(
        _gate_body,
        grid=(nblk,),
        in_specs=[
            pl.BlockSpec((BLK, D), lambda i: (i, 0)),
            pl.BlockSpec((E, D), lambda i: (0, 0)),
        ],
        out_specs=[
            pl.BlockSpec((BLK,), lambda i: (i,)),
            pl.BlockSpec((BLK,), lambda i: (i,)),
            pl.BlockSpec((BLK,), lambda i: (i,)),
            pl.BlockSpec((BLK,), lambda i: (i,)),
        ],
        out_shape=[
            jax.ShapeDtypeStruct((T,), jnp.int32),
            jax.ShapeDtypeStruct((T,), jnp.int32),
            jax.ShapeDtypeStruct((T,), jnp.float32),
            jax.ShapeDtypeStruct((T,), jnp.float32),
        ],
        interpret=_INTERPRET,
    )(inputs, gate_w)


RPW = (T * K) // NW
SLICE = P // NW
_G_CHUNK = 8
_G_NBUF = 6
_G_N = SLICE // _G_CHUNK


def _route_body(e1_hbm, e2_hbm, wa_hbm, wb_hbm, x_hbm,
                w_hbm, pos_hbm, sinfo_hbm, xs_hbm,
                eva_ref, evb_ref, evs_ref, wvs_ref,
                cnt_sh, tok_sh, w_sh,
                allcnt_ref, cntbuf_ref, prefix_ref,
                tokbuf, posbuf, outbuf_f, sinfo_v, idx_v, *grefs):
    rows = list(grefs[:_G_NBUF])
    sg = list(grefs[_G_NBUF:2 * _G_NBUF])
    ss = list(grefs[2 * _G_NBUF:3 * _G_NBUF])
    c = lax.axis_index("c")
    s = lax.axis_index("s")
    lane = lax.broadcasted_iota(jnp.int32, (L,), 0)
    zi = lane * 0
    full_last = zi + (L - 1)

    def oh(e):
        return 1 - jnp.minimum(jnp.abs(lane - e), 1)

    pltpu.sync_copy(e1_hbm.at[pl.ds(s * RPW, RPW)], eva_ref)
    pltpu.sync_copy(e2_hbm.at[pl.ds(s * RPW, RPW)], evb_ref)

    def count_range(ev_ref):
        cnt = zi
        for ch in range(RPW // L):
            ev = ev_ref[pl.ds(ch * L, L)]
            for e in range(E):
                mi = 1 - jnp.minimum(jnp.abs(ev - e), 1)
                pc = jnp.cumsum(mi)
                cnt = cnt + oh(e) * jnp.take(pc, full_last)
        return cnt

    cntbuf_ref[0, pl.ds(0, L)] = count_range(eva_ref)
    pltpu.sync_copy(cntbuf_ref, cnt_sh.at[pl.ds(s, 1)])
    cntbuf_ref[0, pl.ds(0, L)] = count_range(evb_ref)
    pltpu.sync_copy(cntbuf_ref, cnt_sh.at[pl.ds(s + NS, 1)])
    plsc.subcore_barrier()
    pltpu.sync_copy(cnt_sh, allcnt_ref)

    total = zi
    for r in range(NW):
        total = total + allcnt_ref[r, pl.ds(0, L)]
    blocks = (total + (BLK - 1)) >> 8
    cblocks = jnp.cumsum(blocks)
    start = (cblocks - blocks) * BLK

    @pl.when(jnp.logical_and(c == 0, s == 0))
    def _():
        be0 = zi
        be1 = zi
        for e in range(E):
            ce = jnp.take(cblocks, zi + e)
            be0 = be0 + jnp.minimum(jnp.maximum(lane - ce + 1, 0), 1)
            be1 = be1 + jnp.minimum(jnp.maximum(lane + L - ce + 1, 0), 1)
        nbv = jnp.take(cblocks, zi + (E - 1))
        oh_nb = oh(NB - L)
        sinfo_v[pl.ds(0, L)] = be0
        sinfo_v[pl.ds(L, L)] = oh_nb * nbv + (1 - oh_nb) * be1
        pltpu.sync_copy(sinfo_v, sinfo_hbm)

    for d in range(2):
        ehbm = (e1_hbm, e2_hbm)[d]
        whbm = (wa_hbm, wb_hbm)[d]
        pltpu.sync_copy(ehbm.at[pl.ds(s * RPW, RPW)], evs_ref)
        pltpu.sync_copy(whbm.at[pl.ds(s * RPW, RPW)], wvs_ref)

        prefix_ref[...] = zi
        for r in range(NW):
            @pl.when(r < s + d * NS)
            def _():
                prefix_ref[...] = prefix_ref[...] + allcnt_ref[r, pl.ds(0, L)]

        rcur = start + prefix_ref[...]
        for ch in range(RPW // L):
            ev = evs_ref[pl.ds(ch * L, L)]
            base = jnp.take(rcur, ev)
            rank = zi
            hist = zi
            for e in range(E):
                mi = 1 - jnp.minimum(jnp.abs(ev - e), 1)
                pc = jnp.cumsum(mi)
                rank = rank + mi * (pc - mi)
                hist = hist + oh(e) * jnp.take(pc, full_last)
            posv = base + rank
            posv = jnp.minimum(jnp.maximum(posv, 0), P - 1)
            posbuf[pl.ds(ch * L, L)] = posv
            tokbuf[pl.ds(ch * L, L)] = (s * RPW + ch * L) + lane
            rcur = rcur + hist
        pltpu.sync_copy(tokbuf, tok_sh.at[posbuf])
        pltpu.sync_copy(wvs_ref, w_sh.at[posbuf])

        @pl.when(c == 0)
        def _():
            pltpu.sync_copy(posbuf, pos_hbm.at[pl.ds(d * T + s * RPW, RPW)])

    plsc.subcore_barrier()
    j = c * NS + s
    pltpu.sync_copy(w_sh.at[pl.ds(j * SLICE, SLICE)], outbuf_f)
    pltpu.sync_copy(outbuf_f, w_hbm.at[pl.ds(j * SLICE, SLICE)])

    base = j * SLICE
    pltpu.sync_copy(tok_sh.at[pl.ds(base, SLICE)], idx_v)
    for q in range(SLICE // L):
        v = idx_v[pl.ds(q * L, L)]
        idx_v[pl.ds(q * L, L)] = jnp.minimum(jnp.maximum(v, 0), T - 1)

    nvalid = jnp.maximum(jnp.max(cblocks) * BLK - base, 0)

    def g_desc(k):
        b = k % _G_NBUF
        return pltpu.make_async_copy(
            x_hbm.at[idx_v.at[pl.ds(k * _G_CHUNK, _G_CHUNK)]], rows[b], sg[b])

    def s_desc(k):
        b = k % _G_NBUF
        return pltpu.make_async_copy(
            rows[b], xs_hbm.at[pl.ds(base + k * _G_CHUNK, _G_CHUNK)], ss[b])

    def live(k):
        return k * _G_CHUNK < nvalid

    for k in range(_G_NBUF):
        @pl.when(live(k))
        def _():
            g_desc(k).start()
    for k in range(_G_N):
        @pl.when(live(k))
        def _():
            g_desc(k).wait()
        if k + 1 < _G_N and k + 1 >= _G_NBUF:
            @pl.when(live(k + 1 - _G_NBUF))
            def _():
                s_desc(k + 1 - _G_NBUF).wait()

            @pl.when(live(k + 1))
            def _():
                g_desc(k + 1).start()

        @pl.when(live(k))
        def _():
            s_desc(k).start()
    for k in range(_G_N - _G_NBUF, _G_N):
        @pl.when(live(k))
        def _():
            s_desc(k).wait()


def _route_jnp(e1, e2, wa, wb):
    e_all = jnp.concatenate([e1, e2])
    w_all = jnp.concatenate([wa, wb])
    t_all = jnp.concatenate([jnp.arange(T, dtype=jnp.int32)] * 2)
    onehot = (e_all[:, None] == jnp.arange(E)[None, :]).astype(jnp.int32)
    cnt = jnp.sum(onehot, axis=0)
    blocks = (cnt + BLK - 1) // BLK
    cblocks = jnp.cumsum(blocks)
    nb = cblocks[-1]
    start_blk = jnp.concatenate([jnp.zeros((1,), jnp.int32), cblocks[:-1]])
    rank = jnp.cumsum(onehot, axis=0) - onehot
    pos = start_blk[e_all] * BLK + jnp.take_along_axis(rank, e_all[:, None], 1)[:, 0]
    sorted_token = jnp.zeros((P,), jnp.int32).at[pos].set(t_all)
    sorted_w = jnp.zeros((P,), jnp.float32).at[pos].set(w_all)
    be = jnp.searchsorted(cblocks, jnp.arange(NB, dtype=jnp.int32), side="right")
    be = jnp.clip(be, 0, E - 1).astype(jnp.int32)
    sinfo = jnp.concatenate([be, jnp.zeros((L - E,), jnp.int32),
                             nb[None].astype(jnp.int32),
                             jnp.zeros((L - E - 1,), jnp.int32)])
    sinfo = sinfo.at[0:NB].set(be).at[NB].set(nb)
    return sorted_token, sorted_w, pos, sinfo


def _route(e1, e2, wa, wb, inputs):
    mesh = plsc.VectorSubcoreMesh(core_axis_name="c", subcore_axis_name="s")
    f = pl.kernel(
        _route_body,
        out_type=[
            jax.ShapeDtypeStruct((P,), jnp.float32),
            jax.ShapeDtypeStruct((T * K,), jnp.int32),
            jax.ShapeDtypeStruct((2 * L,), jnp.int32),
            jax.ShapeDtypeStruct((P, D), jnp.float32),
        ],
        mesh=mesh,
        compiler_params=pltpu.CompilerParams(needs_layout_passes=False),
        scratch_types=(
            [
                pltpu.VMEM((RPW,), jnp.int32),
                pltpu.VMEM((RPW,), jnp.int32),
                pltpu.VMEM((RPW,), jnp.int32),
                pltpu.VMEM((RPW,), jnp.float32),
                pltpu.VMEM_SHARED((NW, L), jnp.int32),
                pltpu.VMEM_SHARED((P,), jnp.int32),
                pltpu.VMEM_SHARED((P,), jnp.float32),
                pltpu.VMEM((NW, L), jnp.int32),
                pltpu.VMEM((1, L), jnp.int32),
                pltpu.VMEM((L,), jnp.int32),
                pltpu.VMEM((RPW,), jnp.int32),
                pltpu.VMEM((RPW,), jnp.int32),
                pltpu.VMEM((SLICE,), jnp.float32),
                pltpu.VMEM((2 * L,), jnp.int32),
                pltpu.VMEM((SLICE,), jnp.int32),
            ]
            + [pltpu.VMEM((_G_CHUNK, D), jnp.float32)] * _G_NBUF
            + [pltpu.SemaphoreType.DMA] * (2 * _G_NBUF)
        ),
    )
    return f(e1, e2, wa, wb, inputs)


def _ffn1_body(s_ref, xs_ref, w0_ref, w2_ref, h_ref):
    i = pl.program_id(0)
    nb = s_ref[NB]

    @pl.when(i < nb)
    def _():
        x = xs_ref[...].astype(jnp.bfloat16)
        w0 = w0_ref[0, 0].astype(jnp.bfloat16)
        w2 = w2_ref[0, 0].astype(jnp.bfloat16)
        a = jax.lax.dot_general(x, w0, (((1,), (1,)), ((), ())),
                                preferred_element_type=jnp.float32)
        b = jax.lax.dot_general(x, w2, (((1,), (1,)), ((), ())),
                                preferred_element_type=jnp.float32)
        h_ref[...] = (a * jax.lax.logistic(a) * b).astype(jnp.bfloat16)


def _ffn2_body(s_ref, h_ref, w1_ref, wt_ref, y_ref):
    i = pl.program_id(0)
    nb = s_ref[NB]

    @pl.when(i < nb)
    def _():
        w1 = w1_ref[0, 0].astype(jnp.bfloat16)
        part = jax.lax.dot_general(h_ref[...], w1, (((1,), (0,)), ((), ())),
                                   preferred_element_type=jnp.float32)
        y_ref[...] = part * wt_ref[0, 0, :][:, None]


def _ffn(sinfo, xs, expert_ws, sorted_w):
    wt3 = sorted_w.reshape(NB, 1, BLK)

    def emap(slot):
        def f(i, s):
            return (jnp.clip(s[jnp.minimum(i, s[NB] - 1)], 0, E - 1), slot, 0, 0)
        return f

    h = pl.pallas_call(
        _ffn1_body,
        grid_spec=pltpu.PrefetchScalarGridSpec(
            num_scalar_prefetch=1,
            grid=(NB,),
            in_specs=[
                pl.BlockSpec((BLK, D),
                             lambda i, s: (jnp.minimum(i, s[NB] - 1), 0)),
                pl.BlockSpec((1, 1, DFF, D), emap(0)),
                pl.BlockSpec((1, 1, DFF, D), emap(2)),
            ],
            out_specs=pl.BlockSpec((BLK, DFF), lambda i, s: (i, 0)),
        ),
        out_shape=jax.ShapeDtypeStruct((P, DFF), jnp.bfloat16),
        compiler_params=pltpu.CompilerParams(
            dimension_semantics=("arbitrary",),
            vmem_limit_bytes=60 * 1024 * 1024,
        ),
        interpret=_INTERPRET,
    )(sinfo, xs, expert_ws, expert_ws)

    return pl.pallas_call(
        _ffn2_body,
        grid_spec=pltpu.PrefetchScalarGridSpec(
            num_scalar_prefetch=1,
            grid=(NB,),
            in_specs=[
                pl.BlockSpec((BLK, DFF),
                             lambda i, s: (jnp.minimum(i, s[NB] - 1), 0)),
                pl.BlockSpec((1, 1, DFF, D), emap(1)),
                pl.BlockSpec((1, 1, BLK), lambda i, s: (i, 0, 0)),
            ],
            out_specs=pl.BlockSpec((BLK, D), lambda i, s: (i, 0)),
        ),
        out_shape=jax.ShapeDtypeStruct((P, D), jnp.float32),
        compiler_params=pltpu.CompilerParams(
            dimension_semantics=("arbitrary",),
            vmem_limit_bytes=60 * 1024 * 1024,
        ),
        interpret=_INTERPRET,
    )(sinfo, h, expert_ws, wt3)


_C_CHUNK = 8
_C_NBUF = 4
_C_N = T // NW // _C_CHUNK
_TPW = T // NW

def _combine_body(pos_hbm, yw_hbm, out_hbm, i0_v, i1_v, *refs):
    r0 = list(refs[:_C_NBUF])
    r1 = list(refs[_C_NBUF:2 * _C_NBUF])
    sg = list(refs[2 * _C_NBUF:3 * _C_NBUF])
    ss = list(refs[3 * _C_NBUF:4 * _C_NBUF])
    wid = lax.axis_index("s") * NC + lax.axis_index("c")
    base = wid * _TPW
    pltpu.sync_copy(pos_hbm.at[pl.ds(base, _TPW)], i0_v)
    pltpu.sync_copy(pos_hbm.at[pl.ds(T + base, _TPW)], i1_v)

    def g_descs(k):
        b = k % _C_NBUF
        sl = pl.ds(k * _C_CHUNK, _C_CHUNK)
        return (pltpu.make_async_copy(yw_hbm.at[i0_v.at[sl]], r0[b], sg[b]),
                pltpu.make_async_copy(yw_hbm.at[i1_v.at[sl]], r1[b], sg[b]))

    def s_desc(k):
        b = k % _C_NBUF
        return pltpu.make_async_copy(
            r0[b], out_hbm.at[pl.ds(base + k * _C_CHUNK, _C_CHUNK)], ss[b])

    for k in range(_C_NBUF):
        d0, d1 = g_descs(k)
        d0.start()
        d1.start()
    for k in range(_C_N):
        b = k % _C_NBUF
        d0, d1 = g_descs(k)
        d0.wait()
        d1.wait()
        if k + 1 < _C_N and k + 1 >= _C_NBUF:
            s_desc(k + 1 - _C_NBUF).wait()
            n0, n1 = g_descs(k + 1)
            n0.start()
            n1.start()

        def add_body(r, carry):
            for jj in range(D // L):
                csl = pl.ds(jj * L, L)
                r0[b][r, csl] = r0[b][r, csl] + r1[b][r, csl]
            return carry
        lax.fori_loop(0, _C_CHUNK, add_body, 0)
        s_desc(k).start()
    for k in range(_C_N - _C_NBUF, _C_N):
        s_desc(k).wait()


def _combine(yw, pos):
    mesh = plsc.VectorSubcoreMesh(core_axis_name="c", subcore_axis_name="s")
    f = pl.kernel(
        _combine_body,
        out_type=jax.ShapeDtypeStruct((T, D), jnp.float32),
        mesh=mesh,
        scratch_types=(
            [pltpu.VMEM((_TPW,), jnp.int32), pltpu.VMEM((_TPW,), jnp.int32)]
            + [pltpu.VMEM((_C_CHUNK, D), jnp.float32)] * (2 * _C_NBUF)
            + [pltpu.SemaphoreType.DMA] * (2 * _C_NBUF)
        ),
    )
    return f(pos, yw)


def kernel(inputs, gate_w, expert_ws):
    e1, e2, wa, wb = _gate(inputs, gate_w)
    sorted_w, pos, sinfo, xs = _route(e1, e2, wa, wb, inputs)
    yw = _ffn(sinfo, xs, expert_ws, sorted_w)
    return _combine(yw, pos)
